# baseline (speedup 1.0000x reference)
import functools

import jax
import jax.numpy as jnp
from jax.experimental import pallas as pl
from jax.experimental.pallas import tpu as pltpu
from jax.experimental.pallas import tpu_sc as plsc

NC = 2
NS = 16
NW = NC * NS
BR = 8
NLANES = 16


def _sc_body(S, D, x_hbm, pos_hbm, o_hbm):
    def block_body(x_vmem, p_vmem, o_vmem):
        for r in range(BR):
            for j in range(D // NLANES):
                slc = (pl.ds(r, 1), pl.ds(j * NLANES, NLANES))
                o_vmem.at[slc][...] = x_vmem.at[slc][...] + p_vmem.at[slc][...]

    R = x_hbm.shape[0]
    spb = S // BR
    pltpu.emit_pipeline(
        block_body,
        grid=(R // BR,),
        in_specs=[
            pl.BlockSpec((BR, D), index_map=lambda i: (i, 0)),
            pl.BlockSpec((BR, D), index_map=lambda i: (jax.lax.rem(i, spb), 0)),
        ],
        out_specs=[pl.BlockSpec((BR, D), index_map=lambda i: (i, 0))],
        core_axis_name=("c", "s"),
        dimension_semantics=(pltpu.PARALLEL,),
    )(x_hbm, pos_hbm, o_hbm)


@functools.lru_cache(maxsize=None)
def _make_sc_call(B, S, D):
    R = B * S
    mesh = plsc.VectorSubcoreMesh(core_axis_name="c", subcore_axis_name="s")
    return pl.kernel(
        functools.partial(_sc_body, S, D),
        out_type=jax.ShapeDtypeStruct((R, D), jnp.float32),
        mesh=mesh,
    )


def kernel(x, pos_table):
    B, S, D = x.shape
    xf = x.reshape(B * S, D)
    out = _make_sc_call(B, S, D)(xf, pos_table)
    return out.reshape(B, S, D)
